# baseline (speedup 1.0000x reference)
import functools

import jax
import jax.numpy as jnp
from jax import lax
from jax.experimental import pallas as pl
from jax.experimental.pallas import tpu as pltpu
from jax.experimental.pallas import tpu_sc as plsc

D_MODEL = 768
SEQ = 8192
BATCH = 4

NC = 2
NS = 16
NW = NC * NS

ROWS_PER_W = SEQ // NW
CH = 32
CHW = CH * D_MODEL
N_CHUNKS = ROWS_PER_W // CH
UNROLL = 8


def _body(x_hbm, pos_hbm, out_hbm, x_buf, pos_buf):
    w = lax.axis_index("s") * NC + lax.axis_index("c")
    base = (w * ROWS_PER_W) * D_MODEL

    for c in range(N_CHUNKS):
        pos_off = base + c * CHW
        pltpu.sync_copy(pos_hbm.at[pl.ds(pos_off, CHW)], pos_buf)
        for b in range(BATCH):
            x_off = b * (SEQ * D_MODEL) + pos_off
            pltpu.sync_copy(x_hbm.at[pl.ds(x_off, CHW)], x_buf)

            def add_body(i, _):
                o = i * (16 * UNROLL)
                for k in range(UNROLL):
                    sl = pl.ds(o + k * 16, 16)
                    x_buf[sl] = x_buf[sl] + pos_buf[sl]
                return 0

            lax.fori_loop(0, CHW // (16 * UNROLL), add_body, 0)
            pltpu.sync_copy(x_buf, out_hbm.at[pl.ds(x_off, CHW)])


@jax.jit
def kernel(x, pos_embedding):
    seq = x.shape[1]
    x_flat = x.reshape(-1)
    pos_flat = pos_embedding[:seq].reshape(-1)
    mesh = plsc.VectorSubcoreMesh(core_axis_name="c", subcore_axis_name="s")
    out = pl.kernel(
        _body,
        mesh=mesh,
        out_type=jax.ShapeDtypeStruct((x_flat.size,), jnp.float32),
        scratch_types=[
            pltpu.VMEM((CHW,), jnp.float32),
            pltpu.VMEM((CHW,), jnp.float32),
        ],
    )(x_flat, pos_flat)
    return out.reshape(x.shape)


# speedup vs baseline: 1.2593x; 1.2593x over previous
import jax
import jax.numpy as jnp
from jax import lax
from jax.experimental import pallas as pl
from jax.experimental.pallas import tpu as pltpu
from jax.experimental.pallas import tpu_sc as plsc

D_MODEL = 768
SEQ = 8192
BATCH = 4
SD = SEQ * D_MODEL

NC = 2
NS = 16
NW = NC * NS

ROWS_PER_W = SEQ // NW
CH = 8
CHW = CH * D_MODEL
N_CHUNKS = ROWS_PER_W // CH
G = 8
N_GROUPS = CHW // (16 * G)


def _body(x_hbm, pos_hbm, out_hbm,
          xb0, xb1, ob0, ob1, pb0, pb1,
          sx0, sx1, so0, so1, sp0, sp1):
    xb = (xb0, xb1)
    ob = (ob0, ob1)
    pb = (pb0, pb1)
    sx = (sx0, sx1)
    so = (so0, so1)
    sp = (sp0, sp1)

    w = lax.axis_index("s") * NC + lax.axis_index("c")
    base = (w * ROWS_PER_W) * D_MODEL

    def start_in(c, slot):
        off = base + c * CHW
        pltpu.async_copy(pos_hbm.at[pl.ds(off, CHW)], pb[slot], sp[slot])
        for b in range(BATCH):
            pltpu.async_copy(x_hbm.at[pl.ds(b * SD + off, CHW)],
                             xb[slot].at[pl.ds(b * CHW, CHW)], sx[slot])

    def wait_in(c, slot):
        off = base + c * CHW
        pltpu.make_async_copy(pos_hbm.at[pl.ds(off, CHW)],
                              pb[slot], sp[slot]).wait()
        for b in range(BATCH):
            pltpu.make_async_copy(x_hbm.at[pl.ds(b * SD + off, CHW)],
                                  xb[slot].at[pl.ds(b * CHW, CHW)],
                                  sx[slot]).wait()

    def start_out(c, slot):
        off = base + c * CHW
        for b in range(BATCH):
            pltpu.async_copy(ob[slot].at[pl.ds(b * CHW, CHW)],
                             out_hbm.at[pl.ds(b * SD + off, CHW)], so[slot])

    def wait_out(c, slot):
        off = base + c * CHW
        for b in range(BATCH):
            pltpu.make_async_copy(ob[slot].at[pl.ds(b * CHW, CHW)],
                                  out_hbm.at[pl.ds(b * SD + off, CHW)],
                                  so[slot]).wait()

    def compute(slot):
        xs, os_, ps = xb[slot], ob[slot], pb[slot]

        @plsc.parallel_loop(0, N_GROUPS)
        def _(i):
            gbase = i * (16 * G)
            pos_vecs = [ps[pl.ds(gbase + k * 16, 16)] for k in range(G)]
            for b in range(BATCH):
                for k in range(G):
                    sl = pl.ds(b * CHW + gbase + k * 16, 16)
                    os_[sl] = xs[sl] + pos_vecs[k]

    start_in(0, 0)
    start_in(1, 1)
    for c in (0, 1):
        wait_in(c, c)
        compute(c)
        start_out(c, c)
        start_in(c + 2, c)

    @pl.loop(2, N_CHUNKS, step=2)
    def _(c0):
        for d in range(2):
            c = c0 + d
            wait_in(c, d)
            wait_out(c - 2, d)
            compute(d)
            start_out(c, d)

            @pl.when(c + 2 < N_CHUNKS)
            def _():
                start_in(c + 2, d)

    wait_out(N_CHUNKS - 2, 0)
    wait_out(N_CHUNKS - 1, 1)


@jax.jit
def kernel(x, pos_embedding):
    seq = x.shape[1]
    x_flat = x.reshape(-1)
    pos_flat = pos_embedding[:seq].reshape(-1)
    mesh = plsc.VectorSubcoreMesh(core_axis_name="c", subcore_axis_name="s")
    out = pl.kernel(
        _body,
        mesh=mesh,
        out_type=jax.ShapeDtypeStruct((x_flat.size,), jnp.float32),
        scratch_types=[
            pltpu.VMEM((BATCH * CHW,), jnp.float32),
            pltpu.VMEM((BATCH * CHW,), jnp.float32),
            pltpu.VMEM((BATCH * CHW,), jnp.float32),
            pltpu.VMEM((BATCH * CHW,), jnp.float32),
            pltpu.VMEM((CHW,), jnp.float32),
            pltpu.VMEM((CHW,), jnp.float32),
            pltpu.SemaphoreType.DMA,
            pltpu.SemaphoreType.DMA,
            pltpu.SemaphoreType.DMA,
            pltpu.SemaphoreType.DMA,
            pltpu.SemaphoreType.DMA,
            pltpu.SemaphoreType.DMA,
        ],
    )(x_flat, pos_flat)
    return out.reshape(x.shape)


# speedup vs baseline: 3.1272x; 2.4832x over previous
import jax
import jax.numpy as jnp
from jax import lax
from jax.experimental import pallas as pl
from jax.experimental.pallas import tpu as pltpu
from jax.experimental.pallas import tpu_sc as plsc

D_MODEL = 768
SEQ = 8192
BATCH = 4

NC = 2
NS = 16
NW = NC * NS

ROWS_PER_W = SEQ // NW
CH = 8
N_CHUNKS = ROWS_PER_W // CH
G = 6
GROUPS_PER_ROW = D_MODEL // (16 * G)
GW = 16 * G


def _body(x_hbm, pos_hbm, out_hbm,
          xb0, xb1, ob0, ob1, pb0, pb1,
          sx0, sx1, so0, so1, sp0, sp1):
    xb = (xb0, xb1)
    ob = (ob0, ob1)
    pb = (pb0, pb1)
    sx = (sx0, sx1)
    so = (so0, so1)
    sp = (sp0, sp1)

    w = lax.axis_index("s") * NC + lax.axis_index("c")
    base = w * ROWS_PER_W

    def start_in(c, slot):
        r0 = base + c * CH
        pltpu.async_copy(pos_hbm.at[pl.ds(r0, CH), :], pb[slot], sp[slot])
        for b in range(BATCH):
            pltpu.async_copy(x_hbm.at[b, pl.ds(r0, CH), :],
                             xb[slot].at[b], sx[slot])

    def wait_in(c, slot):
        r0 = base + c * CH
        pltpu.make_async_copy(pos_hbm.at[pl.ds(r0, CH), :],
                              pb[slot], sp[slot]).wait()
        for b in range(BATCH):
            pltpu.make_async_copy(x_hbm.at[b, pl.ds(r0, CH), :],
                                  xb[slot].at[b], sx[slot]).wait()

    def start_out(c, slot):
        r0 = base + c * CH
        for b in range(BATCH):
            pltpu.async_copy(ob[slot].at[b],
                             out_hbm.at[b, pl.ds(r0, CH), :], so[slot])

    def wait_out(c, slot):
        r0 = base + c * CH
        for b in range(BATCH):
            pltpu.make_async_copy(ob[slot].at[b],
                                  out_hbm.at[b, pl.ds(r0, CH), :],
                                  so[slot]).wait()

    def compute(slot):
        xs, os_, ps = xb[slot], ob[slot], pb[slot]

        @plsc.parallel_loop(0, CH * GROUPS_PER_ROW)
        def _(i):
            row = i // GROUPS_PER_ROW
            c0 = (i % GROUPS_PER_ROW) * GW
            pos_vecs = [ps[row, pl.ds(c0 + k * 16, 16)] for k in range(G)]
            for b in range(BATCH):
                for k in range(G):
                    sl = pl.ds(c0 + k * 16, 16)
                    os_[b, row, sl] = xs[b, row, sl] + pos_vecs[k]

    start_in(0, 0)
    start_in(1, 1)
    for c in (0, 1):
        wait_in(c, c)
        compute(c)
        start_out(c, c)
        start_in(c + 2, c)

    @pl.loop(2, N_CHUNKS, step=2)
    def _(c0):
        for d in range(2):
            c = c0 + d
            wait_in(c, d)
            wait_out(c - 2, d)
            compute(d)
            start_out(c, d)

            @pl.when(c + 2 < N_CHUNKS)
            def _():
                start_in(c + 2, d)

    wait_out(N_CHUNKS - 2, 0)
    wait_out(N_CHUNKS - 1, 1)


@jax.jit
def kernel(x, pos_embedding):
    seq = x.shape[1]
    pos = pos_embedding[:seq]
    mesh = plsc.VectorSubcoreMesh(core_axis_name="c", subcore_axis_name="s")
    return pl.kernel(
        _body,
        mesh=mesh,
        out_type=jax.ShapeDtypeStruct(x.shape, jnp.float32),
        scratch_types=[
            pltpu.VMEM((BATCH, CH, D_MODEL), jnp.float32),
            pltpu.VMEM((BATCH, CH, D_MODEL), jnp.float32),
            pltpu.VMEM((BATCH, CH, D_MODEL), jnp.float32),
            pltpu.VMEM((BATCH, CH, D_MODEL), jnp.float32),
            pltpu.VMEM((CH, D_MODEL), jnp.float32),
            pltpu.VMEM((CH, D_MODEL), jnp.float32),
            pltpu.SemaphoreType.DMA,
            pltpu.SemaphoreType.DMA,
            pltpu.SemaphoreType.DMA,
            pltpu.SemaphoreType.DMA,
            pltpu.SemaphoreType.DMA,
            pltpu.SemaphoreType.DMA,
        ],
    )(x, pos)
